# baseline (speedup 1.0000x reference)
import functools

import jax
import jax.numpy as jnp
from jax import lax
from jax.experimental import pallas as pl
from jax.experimental.pallas import tpu as pltpu

_N_DEVICES = 8
_OUT_DIM = 64
_B = 4096
_TABLE = (16, 24, 32, 40, 48, 56, 60, 64)
_ROWS_PER_BLK = 512
_GRID = _B // _ROWS_PER_BLK


def _mse_body(ids_ref, y_ref, yp_ref, out_ref):
    step = pl.program_id(0)

    ids = ids_ref[...].astype(jnp.int32)
    nv = jnp.zeros_like(ids)
    for d in range(_N_DEVICES):
        nv = jnp.where(ids == d, _TABLE[d], nv)

    col = lax.broadcasted_iota(jnp.int32, (_ROWS_PER_BLK, _OUT_DIM), 1)
    valid = col < nv
    diff = y_ref[...] - yp_ref[...]
    part = jnp.sum(jnp.where(valid, diff * diff, 0.0))

    @pl.when(step == 0)
    def _():
        out_ref[0, 0] = 0.0

    out_ref[0, 0] += part

    @pl.when(step == _GRID - 1)
    def _():
        out_ref[0, 0] = out_ref[0, 0] * (1.0 / (_B * _OUT_DIM))


@jax.jit
def _masked_mse(ids_f, y, y_pred):
    out = pl.pallas_call(
        _mse_body,
        grid=(_GRID,),
        in_specs=[
            pl.BlockSpec((_ROWS_PER_BLK, 1), lambda i: (i, 0)),
            pl.BlockSpec((_ROWS_PER_BLK, _OUT_DIM), lambda i: (i, 0)),
            pl.BlockSpec((_ROWS_PER_BLK, _OUT_DIM), lambda i: (i, 0)),
        ],
        out_specs=pl.BlockSpec(
            (1, 1), lambda i: (0, 0), memory_space=pltpu.SMEM
        ),
        out_shape=jax.ShapeDtypeStruct((1, 1), jnp.float32),
    )(ids_f, y, y_pred)
    return out[0, 0]


def kernel(x, y, y_pred):
    ids_f = x[:, 0, 0].reshape(_B, 1)
    return _masked_mse(ids_f, y, y_pred)


# speedup vs baseline: 1.0416x; 1.0416x over previous
import functools

import jax
import jax.numpy as jnp
from jax import lax
from jax.experimental import pallas as pl
from jax.experimental.pallas import tpu as pltpu

_N_DEVICES = 8
_OUT_DIM = 64
_B = 4096
_TABLE = (16, 24, 32, 40, 48, 56, 60, 64)
_ROWS_PER_BLK = 512
_GRID = _B // _ROWS_PER_BLK


def _mse_body(ids_ref, y_ref, yp_ref, out_ref):
    step = pl.program_id(0)

    idb = jnp.broadcast_to(ids_ref[...], (_ROWS_PER_BLK, _OUT_DIM))
    nv = 16.0 + 8.0 * idb - 4.0 * jnp.maximum(idb - 5.0, 0.0)
    col = lax.broadcasted_iota(
        jnp.int32, (_ROWS_PER_BLK, _OUT_DIM), 1
    ).astype(jnp.float32)
    diff = y_ref[...] - yp_ref[...]
    part = jnp.sum(jnp.where(col < nv, diff * diff, 0.0))

    @pl.when(step == 0)
    def _():
        out_ref[0, 0] = 0.0

    out_ref[0, 0] += part

    @pl.when(step == _GRID - 1)
    def _():
        out_ref[0, 0] = out_ref[0, 0] * (1.0 / (_B * _OUT_DIM))


@jax.jit
def _masked_mse(ids_f, y, y_pred):
    out = pl.pallas_call(
        _mse_body,
        grid=(_GRID,),
        in_specs=[
            pl.BlockSpec((_ROWS_PER_BLK, 1), lambda i: (i, 0)),
            pl.BlockSpec((_ROWS_PER_BLK, _OUT_DIM), lambda i: (i, 0)),
            pl.BlockSpec((_ROWS_PER_BLK, _OUT_DIM), lambda i: (i, 0)),
        ],
        out_specs=pl.BlockSpec(
            (1, 1), lambda i: (0, 0), memory_space=pltpu.SMEM
        ),
        out_shape=jax.ShapeDtypeStruct((1, 1), jnp.float32),
    )(ids_f, y, y_pred)
    return out[0, 0]


def kernel(x, y, y_pred):
    ids_f = x[:, 0, 0].reshape(_B, 1)
    return _masked_mse(ids_f, y, y_pred)


# speedup vs baseline: 1.9973x; 1.9175x over previous
import jax
import jax.numpy as jnp
from jax import lax
from jax.experimental import pallas as pl
from jax.experimental.pallas import tpu as pltpu

_OUT_DIM = 64
_B = 4096
_COLS_PER_BLK = 512
_GRID = _B // _COLS_PER_BLK


def _mse_body(ids_ref, y_ref, yp_ref, out_ref):
    step = pl.program_id(0)

    ids = ids_ref[...].astype(jnp.int32)
    nv = 16 + 8 * ids - 4 * jnp.maximum(ids - 5, 0)
    nvb = jnp.broadcast_to(nv, (_OUT_DIM, _COLS_PER_BLK))
    feat = lax.broadcasted_iota(jnp.int32, (_OUT_DIM, _COLS_PER_BLK), 0)
    diff = y_ref[...] - yp_ref[...]
    part = jnp.sum(jnp.where(feat < nvb, diff * diff, 0.0))

    @pl.when(step == 0)
    def _():
        out_ref[0, 0] = 0.0

    out_ref[0, 0] += part

    @pl.when(step == _GRID - 1)
    def _():
        out_ref[0, 0] = out_ref[0, 0] * (1.0 / (_B * _OUT_DIM))


@jax.jit
def _masked_mse(ids_f, yt, ypt):
    out = pl.pallas_call(
        _mse_body,
        grid=(_GRID,),
        in_specs=[
            pl.BlockSpec((1, _COLS_PER_BLK), lambda i: (0, i)),
            pl.BlockSpec((_OUT_DIM, _COLS_PER_BLK), lambda i: (0, i)),
            pl.BlockSpec((_OUT_DIM, _COLS_PER_BLK), lambda i: (0, i)),
        ],
        out_specs=pl.BlockSpec(
            (1, 1), lambda i: (0, 0), memory_space=pltpu.SMEM
        ),
        out_shape=jax.ShapeDtypeStruct((1, 1), jnp.float32),
    )(ids_f, yt, ypt)
    return out[0, 0]


def kernel(x, y, y_pred):
    ids_f = x[:, 0, 0].reshape(1, _B)
    return _masked_mse(ids_f, y.T, y_pred.T)


# speedup vs baseline: 3.8278x; 1.9164x over previous
import jax
import jax.numpy as jnp
from jax import lax
from jax.experimental import pallas as pl
from jax.experimental.pallas import tpu as pltpu

_OUT_DIM = 64
_B = 4096
_COLS_PER_BLK = 1024
_GRID = _B // _COLS_PER_BLK


def _mse_body(ids_ref, y_ref, yp_ref, out_ref):
    step = pl.program_id(0)

    ids = ids_ref[...].astype(jnp.int32)
    nv = 16 + 8 * ids - 4 * jnp.maximum(ids - 5, 0)
    nvb = jnp.broadcast_to(nv, (_OUT_DIM, _COLS_PER_BLK))
    feat = lax.broadcasted_iota(jnp.int32, (_OUT_DIM, _COLS_PER_BLK), 0)
    diff = y_ref[...] - yp_ref[...]
    part = jnp.sum(jnp.where(feat < nvb, diff * diff, 0.0))

    @pl.when(step == 0)
    def _():
        out_ref[0, 0] = 0.0

    out_ref[0, 0] += part

    @pl.when(step == _GRID - 1)
    def _():
        out_ref[0, 0] = out_ref[0, 0] * (1.0 / (_B * _OUT_DIM))


@jax.jit
def _masked_mse(ids_f, yt, ypt):
    out = pl.pallas_call(
        _mse_body,
        grid=(_GRID,),
        in_specs=[
            pl.BlockSpec((1, _COLS_PER_BLK), lambda i: (0, i)),
            pl.BlockSpec((_OUT_DIM, _COLS_PER_BLK), lambda i: (0, i)),
            pl.BlockSpec((_OUT_DIM, _COLS_PER_BLK), lambda i: (0, i)),
        ],
        out_specs=pl.BlockSpec(
            (1, 1), lambda i: (0, 0), memory_space=pltpu.SMEM
        ),
        out_shape=jax.ShapeDtypeStruct((1, 1), jnp.float32),
        compiler_params=pltpu.CompilerParams(
            allow_input_fusion=[True, False, False],
        ),
    )(ids_f, yt, ypt)
    return out[0, 0]


def kernel(x, y, y_pred):
    ids_f = x[:, 0, 0].reshape(1, _B)
    return _masked_mse(ids_f, y.T, y_pred.T)


# speedup vs baseline: 5.3318x; 1.3929x over previous
import jax
import jax.numpy as jnp
from jax import lax
from jax.experimental import pallas as pl
from jax.experimental.pallas import tpu as pltpu

_OUT_DIM = 64
_B = 4096
_COLS_PER_BLK = 2048
_GRID = _B // _COLS_PER_BLK


def _mse_body(ids_ref, y_ref, yp_ref, out_ref):
    step = pl.program_id(0)

    ids = ids_ref[...].astype(jnp.int32)
    nv = 16 + 8 * ids - 4 * jnp.maximum(ids - 5, 0)
    nvb = jnp.broadcast_to(nv, (_OUT_DIM, _COLS_PER_BLK))
    feat = lax.broadcasted_iota(jnp.int32, (_OUT_DIM, _COLS_PER_BLK), 0)
    diff = y_ref[...] - yp_ref[...]
    part = jnp.sum(jnp.where(feat < nvb, diff * diff, 0.0))

    @pl.when(step == 0)
    def _():
        out_ref[0, 0] = 0.0

    out_ref[0, 0] += part

    @pl.when(step == _GRID - 1)
    def _():
        out_ref[0, 0] = out_ref[0, 0] * (1.0 / (_B * _OUT_DIM))


@jax.jit
def _masked_mse(ids_f, yt, ypt):
    out = pl.pallas_call(
        _mse_body,
        grid=(_GRID,),
        in_specs=[
            pl.BlockSpec((1, _COLS_PER_BLK), lambda i: (0, i)),
            pl.BlockSpec((_OUT_DIM, _COLS_PER_BLK), lambda i: (0, i)),
            pl.BlockSpec((_OUT_DIM, _COLS_PER_BLK), lambda i: (0, i)),
        ],
        out_specs=pl.BlockSpec(
            (1, 1), lambda i: (0, 0), memory_space=pltpu.SMEM
        ),
        out_shape=jax.ShapeDtypeStruct((1, 1), jnp.float32),
        compiler_params=pltpu.CompilerParams(
            allow_input_fusion=[True, False, False],
        ),
    )(ids_f, yt, ypt)
    return out[0, 0]


def kernel(x, y, y_pred):
    ids_f = x[:, 0, 0].reshape(1, _B)
    return _masked_mse(ids_f, y.T, y_pred.T)
